# baseline (speedup 1.0000x reference)
import dataclasses

import jax
import jax.numpy as jnp
from jax import lax
from jax.experimental import pallas as pl
from jax.experimental.pallas import tpu as pltpu
from jax.experimental.pallas import tpu_sc as plsc

N_USERS = 1000000
N_HORIZONS = 16
BATCH = 16384
LANES = 16

NUM_CORES = 2
NUM_SUBCORES = 16
NUM_WORKERS = NUM_CORES * NUM_SUBCORES
IDS_PER_WORKER = BATCH // NUM_WORKERS
CHUNKS = IDS_PER_WORKER // LANES
TR_BLOCK = 8192


def _tc_transpose_kernel(s_ref, b_ref, os_ref, ob_ref):
    os_ref[...] = s_ref[...].T
    ob_ref[...] = b_ref[...].T


def _tc_transpose(st, bt):
    out = jax.ShapeDtypeStruct((N_USERS, N_HORIZONS), jnp.float32)
    return pl.pallas_call(
        _tc_transpose_kernel,
        grid=((N_USERS + TR_BLOCK - 1) // TR_BLOCK,),
        in_specs=[
            pl.BlockSpec((N_HORIZONS, TR_BLOCK), lambda k: (0, k)),
            pl.BlockSpec((N_HORIZONS, TR_BLOCK), lambda k: (0, k)),
        ],
        out_specs=[
            pl.BlockSpec((TR_BLOCK, N_HORIZONS), lambda k: (k, 0)),
            pl.BlockSpec((TR_BLOCK, N_HORIZONS), lambda k: (k, 0)),
        ],
        out_shape=(out, out),
    )(st, bt)


def _sc_gather_kernel(st_hbm, bt_hbm, idx_hbm, s_out, b_out,
                      idx_v, rs_v, rb_v, s_v, b_v, sem_s, sem_b):
    wid = lax.axis_index("s") * NUM_CORES + lax.axis_index("c")
    base = wid * IDS_PER_WORKER
    pltpu.sync_copy(idx_hbm.at[pl.ds(base, IDS_PER_WORKER)], idx_v)
    cs = pltpu.async_copy(st_hbm.at[idx_v], rs_v, sem_s)
    cb = pltpu.async_copy(bt_hbm.at[idx_v], rb_v, sem_b)
    cs.wait()
    cb.wait()
    iota = lax.iota(jnp.int32, LANES)

    @pl.loop(0, CHUNKS)
    def _(k):
        rows16 = iota + k * LANES
        for h in range(N_HORIZONS):
            hvec = jnp.full((LANES,), h, jnp.int32)
            s_v[h, pl.ds(k * LANES, LANES)] = plsc.load_gather(
                rs_v, [rows16, hvec])
            b_v[h, pl.ds(k * LANES, LANES)] = plsc.load_gather(
                rb_v, [rows16, hvec])

    pltpu.sync_copy(s_v, s_out.at[:, pl.ds(base, IDS_PER_WORKER)])
    pltpu.sync_copy(b_v, b_out.at[:, pl.ds(base, IDS_PER_WORKER)])


def _sc_compiler_params():
    cp = pltpu.CompilerParams(use_tc_tiling_on_sc=False)
    if "needs_layout_passes" in pltpu.CompilerParams.__dataclass_fields__:
        cp = dataclasses.replace(cp, needs_layout_passes=False)
    return cp


def _sc_gather(st_rm, bt_rm, idx):
    mesh = plsc.VectorSubcoreMesh(core_axis_name="c", subcore_axis_name="s")
    out = jax.ShapeDtypeStruct((N_HORIZONS, BATCH), jnp.float32)
    kern = pl.kernel(
        _sc_gather_kernel,
        mesh=mesh,
        out_type=(out, out),
        scratch_types=[
            pltpu.VMEM((IDS_PER_WORKER,), jnp.int32),
            pltpu.VMEM((IDS_PER_WORKER, N_HORIZONS), jnp.float32),
            pltpu.VMEM((IDS_PER_WORKER, N_HORIZONS), jnp.float32),
            pltpu.VMEM((N_HORIZONS, IDS_PER_WORKER), jnp.float32),
            pltpu.VMEM((N_HORIZONS, IDS_PER_WORKER), jnp.float32),
            pltpu.SemaphoreType.DMA,
            pltpu.SemaphoreType.DMA,
        ],
        compiler_params=_sc_compiler_params(),
    )
    return kern(st_rm, bt_rm, idx)


def _tc_math_kernel(p_ref, s_ref, b_ref, o_ref):
    eps = 1e-07
    p = jnp.clip(p_ref[...], eps, 1.0 - eps)
    logits = jnp.log(p / (1.0 - p))
    o_ref[...] = jax.nn.sigmoid(logits * s_ref[...] + b_ref[...])


def _tc_math(p2, s2, b2):
    return pl.pallas_call(
        _tc_math_kernel,
        out_shape=jax.ShapeDtypeStruct(p2.shape, jnp.float32),
    )(p2, s2, b2)


@jax.jit
def kernel(probs, user_ids, scale_table, bias_table):
    idx = user_ids.astype(jnp.int32)
    st_rm, bt_rm = _tc_transpose(scale_table.T, bias_table.T)
    scale_g, bias_g = _sc_gather(st_rm, bt_rm, idx)
    out_t = _tc_math(probs.T, scale_g, bias_g)
    return out_t.T


# speedup vs baseline: 1.2478x; 1.2478x over previous
import dataclasses

import jax
import jax.numpy as jnp
from jax import lax
from jax.experimental import pallas as pl
from jax.experimental.pallas import tpu as pltpu
from jax.experimental.pallas import tpu_sc as plsc

N_USERS = 1000000
N_HORIZONS = 16
BATCH = 16384
LANES = 16

NUM_CORES = 2
NUM_SUBCORES = 16
NUM_WORKERS = NUM_CORES * NUM_SUBCORES
IDS_PER_WORKER = BATCH // NUM_WORKERS
CHUNKS = IDS_PER_WORKER // LANES


def _sc_gather_kernel(st_hbm, bt_hbm, idx_hbm, s_out, b_out,
                      idx_v, rs_v, rb_v, s_v, b_v, sem_s, sem_b):
    wid = lax.axis_index("s") * NUM_CORES + lax.axis_index("c")
    base = wid * IDS_PER_WORKER
    pltpu.sync_copy(idx_hbm.at[pl.ds(base, IDS_PER_WORKER)], idx_v)
    cs = pltpu.async_copy(st_hbm.at[idx_v], rs_v, sem_s)
    cb = pltpu.async_copy(bt_hbm.at[idx_v], rb_v, sem_b)
    cs.wait()
    cb.wait()
    iota = lax.iota(jnp.int32, LANES)

    @pl.loop(0, CHUNKS)
    def _(k):
        rows16 = iota + k * LANES
        for h in range(N_HORIZONS):
            hvec = jnp.full((LANES,), h, jnp.int32)
            s_v[h, pl.ds(k * LANES, LANES)] = plsc.load_gather(
                rs_v, [rows16, hvec])
            b_v[h, pl.ds(k * LANES, LANES)] = plsc.load_gather(
                rb_v, [rows16, hvec])

    pltpu.sync_copy(s_v, s_out.at[:, pl.ds(base, IDS_PER_WORKER)])
    pltpu.sync_copy(b_v, b_out.at[:, pl.ds(base, IDS_PER_WORKER)])


def _sc_compiler_params():
    cp = pltpu.CompilerParams(use_tc_tiling_on_sc=False)
    if "needs_layout_passes" in pltpu.CompilerParams.__dataclass_fields__:
        cp = dataclasses.replace(cp, needs_layout_passes=False)
    return cp


def _sc_gather(st_rm, bt_rm, idx):
    mesh = plsc.VectorSubcoreMesh(core_axis_name="c", subcore_axis_name="s")
    out = jax.ShapeDtypeStruct((N_HORIZONS, BATCH), jnp.float32)
    kern = pl.kernel(
        _sc_gather_kernel,
        mesh=mesh,
        out_type=(out, out),
        scratch_types=[
            pltpu.VMEM((IDS_PER_WORKER,), jnp.int32),
            pltpu.VMEM((IDS_PER_WORKER, N_HORIZONS), jnp.float32),
            pltpu.VMEM((IDS_PER_WORKER, N_HORIZONS), jnp.float32),
            pltpu.VMEM((N_HORIZONS, IDS_PER_WORKER), jnp.float32),
            pltpu.VMEM((N_HORIZONS, IDS_PER_WORKER), jnp.float32),
            pltpu.SemaphoreType.DMA,
            pltpu.SemaphoreType.DMA,
        ],
        compiler_params=_sc_compiler_params(),
    )
    return kern(st_rm, bt_rm, idx)


def _tc_math_kernel(p_ref, s_ref, b_ref, o_ref):
    eps = 1e-07
    p = jnp.clip(p_ref[...], eps, 1.0 - eps)
    logits = jnp.log(p / (1.0 - p))
    o_ref[...] = jax.nn.sigmoid(logits * s_ref[...] + b_ref[...])


def _tc_math(p2, s2, b2):
    return pl.pallas_call(
        _tc_math_kernel,
        out_shape=jax.ShapeDtypeStruct(p2.shape, jnp.float32),
    )(p2, s2, b2)


@jax.jit
def kernel(probs, user_ids, scale_table, bias_table):
    idx = user_ids.astype(jnp.int32)
    scale_g, bias_g = _sc_gather(scale_table, bias_table, idx)
    out_t = _tc_math(probs.T, scale_g, bias_g)
    return out_t.T
